# baseline (speedup 1.0000x reference)
import functools

import jax
import jax.numpy as jnp
from jax import lax
from jax.experimental import pallas as pl
from jax.experimental.pallas import tpu as pltpu
from jax.experimental.pallas import tpu_sc as plsc

SRC_SHAPE = (16384, 1024)
ROWS, COLS = SRC_SHAPE
WROWS = ROWS // 4
NUM_WORKERS = 32
ROWS_PER_WORKER = ROWS // NUM_WORKERS
CHUNK_ROWS = 16
CHUNK_WROWS = CHUNK_ROWS // 4
NUM_PAIRS = ROWS_PER_WORKER // (2 * CHUNK_ROWS)
CHUNK = CHUNK_ROWS * COLS

_mesh = plsc.VectorSubcoreMesh(core_axis_name="c", subcore_axis_name="s")


@functools.partial(
    pl.kernel,
    mesh=_mesh,
    compiler_params=pltpu.CompilerParams(needs_layout_passes=False),
    out_type=jax.ShapeDtypeStruct((WROWS, COLS), jnp.int32),
    scratch_types=[
        pltpu.VMEM((256,), jnp.float32),
        pltpu.VMEM((CHUNK_ROWS, COLS), jnp.float32),
        pltpu.VMEM((CHUNK_ROWS, COLS), jnp.float32),
        pltpu.VMEM((CHUNK_ROWS, COLS), jnp.int32),
        pltpu.VMEM((CHUNK_ROWS, COLS), jnp.int32),
        pltpu.VMEM((2 * CHUNK_WROWS, COLS), jnp.int32),
        pltpu.VMEM((2 * CHUNK_WROWS, COLS), jnp.int32),
        pltpu.SemaphoreType.DMA,
        pltpu.SemaphoreType.DMA,
        pltpu.SemaphoreType.DMA,
        pltpu.SemaphoreType.DMA,
        pltpu.SemaphoreType.DMA,
        pltpu.SemaphoreType.DMA,
    ],
)
def _bsgen_sc(src_hbm, table_hbm, idx_hbm, out_hbm,
              table_v, src_v0, src_v1, idx_v0, idx_v1, out_v0, out_v1,
              sem_s0, sem_s1, sem_i0, sem_i1, sem_o0, sem_o1):
    wid = lax.axis_index("s") * 2 + lax.axis_index("c")
    row0 = wid * ROWS_PER_WORKER
    wrow0 = wid * (ROWS_PER_WORKER // 4)

    in_slots = (
        (src_v0, idx_v0, sem_s0, sem_i0),
        (src_v1, idx_v1, sem_s1, sem_i1),
    )
    out_bufs = ((out_v0, sem_o0), (out_v1, sem_o1))

    def start_in(g, sl):
        src_v, idx_v, sem_s, sem_i = sl
        r = row0 + g * CHUNK_ROWS
        pltpu.async_copy(src_hbm.at[pl.ds(r, CHUNK_ROWS), :], src_v, sem_s)
        pltpu.async_copy(idx_hbm.at[pl.ds(r, CHUNK_ROWS), :], idx_v, sem_i)

    def wait_in(sl):
        src_v, idx_v, sem_s, sem_i = sl
        pltpu.make_async_copy(src_hbm.at[pl.ds(row0, CHUNK_ROWS), :], src_v,
                              sem_s).wait()
        pltpu.make_async_copy(idx_hbm.at[pl.ds(row0, CHUNK_ROWS), :], idx_v,
                              sem_i).wait()

    def start_out(p, ob):
        out_v, sem_o = ob
        r = wrow0 + p * (2 * CHUNK_WROWS)
        pltpu.async_copy(out_v, out_hbm.at[pl.ds(r, 2 * CHUNK_WROWS), :], sem_o)

    def wait_out(ob):
        out_v, sem_o = ob
        pltpu.make_async_copy(out_v,
                              out_hbm.at[pl.ds(wrow0, 2 * CHUNK_WROWS), :],
                              sem_o).wait()

    pltpu.sync_copy(table_hbm, table_v)

    NCG = COLS // 16

    def compute(sl, out_v, half):
        src_v, idx_v = sl[0], sl[1]
        wr_base = half * CHUNK_WROWS

        for wr in range(CHUNK_WROWS):

            @plsc.parallel_loop(0, NCG, 1, unroll=8)
            def inner(j):
                c0 = j * 16
                acc = None
                for b in range(4):
                    r = 4 * wr + b
                    iv = idx_v[r, pl.ds(c0, 16)]
                    tv = plsc.load_gather(table_v, [iv])
                    sv = src_v[r, pl.ds(c0, 16)]
                    rm = jnp.where(sv > tv, jnp.int32(1 << (8 * b)),
                                   jnp.int32(0))
                    acc = rm if acc is None else acc | rm
                out_v[wr_base + wr, pl.ds(c0, 16)] = acc

    for b in range(2):
        start_in(b, in_slots[b])

    def quad_body(q, carry):
        for pb in range(2):
            p = q * 2 + pb
            ob = out_bufs[pb]

            @pl.when(p >= 2)
            def _():
                wait_out(ob)

            for b in range(2):
                g = p * 2 + b
                sl = in_slots[b]
                wait_in(sl)
                compute(sl, ob[0], b)

                @pl.when(g + 2 < 2 * NUM_PAIRS)
                def _():
                    start_in(g + 2, sl)

            start_out(p, ob)
        return carry

    lax.fori_loop(0, NUM_PAIRS // 2, quad_body, 0)

    for pb in range(2):
        wait_out(out_bufs[pb])


_TC_BLOCK_WROWS = 512


def _unpack_body(w_ref, o_ref):
    o_ref[...] = pltpu.bitcast(w_ref[...], jnp.int8)


_unpack_tc = pl.pallas_call(
    _unpack_body,
    grid=(WROWS // _TC_BLOCK_WROWS,),
    in_specs=[pl.BlockSpec((_TC_BLOCK_WROWS, COLS), lambda i: (i, 0))],
    out_specs=pl.BlockSpec((4 * _TC_BLOCK_WROWS, COLS), lambda i: (i, 0)),
    out_shape=jax.ShapeDtypeStruct(SRC_SHAPE, jnp.int8),
)


def kernel(source, rng_seq, rng_idx):
    idx = rng_idx.astype(jnp.int32)
    out_w = _bsgen_sc(source, rng_seq, idx)
    return _unpack_tc(out_w)


# speedup vs baseline: 1.1205x; 1.1205x over previous
import functools

import jax
import jax.numpy as jnp
from jax import lax
from jax.experimental import pallas as pl
from jax.experimental.pallas import tpu as pltpu
from jax.experimental.pallas import tpu_sc as plsc

SRC_SHAPE = (16384, 1024)
ROWS, COLS = SRC_SHAPE
WROWS = ROWS // 4
NUM_WORKERS = 32
ROWS_PER_WORKER = ROWS // NUM_WORKERS
CHUNK_ROWS = 16
CHUNK_WROWS = CHUNK_ROWS // 4
NUM_PAIRS = ROWS_PER_WORKER // (2 * CHUNK_ROWS)
CHUNK = CHUNK_ROWS * COLS

_mesh = plsc.VectorSubcoreMesh(core_axis_name="c", subcore_axis_name="s")


@functools.partial(
    pl.kernel,
    mesh=_mesh,
    compiler_params=pltpu.CompilerParams(needs_layout_passes=False),
    out_type=jax.ShapeDtypeStruct((WROWS, COLS), jnp.int32),
    scratch_types=[
        pltpu.VMEM((256,), jnp.float32),
        pltpu.VMEM((CHUNK_ROWS, COLS), jnp.float32),
        pltpu.VMEM((CHUNK_ROWS, COLS), jnp.float32),
        pltpu.VMEM((CHUNK_ROWS, COLS), jnp.int32),
        pltpu.VMEM((CHUNK_ROWS, COLS), jnp.int32),
        pltpu.VMEM((2 * CHUNK_WROWS, COLS), jnp.int32),
        pltpu.VMEM((2 * CHUNK_WROWS, COLS), jnp.int32),
        pltpu.SemaphoreType.DMA,
        pltpu.SemaphoreType.DMA,
        pltpu.SemaphoreType.DMA,
        pltpu.SemaphoreType.DMA,
        pltpu.SemaphoreType.DMA,
        pltpu.SemaphoreType.DMA,
    ],
)
def _bsgen_sc(src_hbm, table_hbm, idx_hbm, out_hbm,
              table_v, src_v0, src_v1, idx_v0, idx_v1, out_v0, out_v1,
              sem_s0, sem_s1, sem_i0, sem_i1, sem_o0, sem_o1):
    wid = lax.axis_index("s") * 2 + lax.axis_index("c")
    row0 = wid * ROWS_PER_WORKER
    wrow0 = wid * (ROWS_PER_WORKER // 4)

    in_slots = (
        (src_v0, idx_v0, sem_s0, sem_i0),
        (src_v1, idx_v1, sem_s1, sem_i1),
    )
    out_bufs = ((out_v0, sem_o0), (out_v1, sem_o1))

    def start_in(g, sl):
        src_v, idx_v, sem_s, sem_i = sl
        r = row0 + g * CHUNK_ROWS
        pltpu.async_copy(src_hbm.at[pl.ds(r, CHUNK_ROWS), :], src_v, sem_s)
        pltpu.async_copy(idx_hbm.at[pl.ds(r, CHUNK_ROWS), :], idx_v, sem_i)

    def wait_in(sl):
        src_v, idx_v, sem_s, sem_i = sl
        pltpu.make_async_copy(src_hbm.at[pl.ds(row0, CHUNK_ROWS), :], src_v,
                              sem_s).wait()
        pltpu.make_async_copy(idx_hbm.at[pl.ds(row0, CHUNK_ROWS), :], idx_v,
                              sem_i).wait()

    def start_out(p, ob):
        out_v, sem_o = ob
        r = wrow0 + p * (2 * CHUNK_WROWS)
        pltpu.async_copy(out_v, out_hbm.at[pl.ds(r, 2 * CHUNK_WROWS), :], sem_o)

    def wait_out(ob):
        out_v, sem_o = ob
        pltpu.make_async_copy(out_v,
                              out_hbm.at[pl.ds(wrow0, 2 * CHUNK_WROWS), :],
                              sem_o).wait()

    pltpu.sync_copy(table_hbm, table_v)

    NCG = COLS // 16

    def compute(sl, out_v, half):
        src_v, idx_v = sl[0], sl[1]
        wr_base = half * CHUNK_WROWS

        @plsc.parallel_loop(0, CHUNK // 64, 1, unroll=8)
        def inner(j):
            wr = j // NCG
            c0 = (j % NCG) * 16
            acc = None
            for b in range(4):
                r = 4 * wr + b
                iv = idx_v[r, pl.ds(c0, 16)]
                tv = plsc.load_gather(table_v, [iv])
                sv = src_v[r, pl.ds(c0, 16)]
                rm = jnp.where(sv > tv, jnp.int32(1 << (8 * b)), jnp.int32(0))
                acc = rm if acc is None else acc | rm
            out_v[wr_base + wr, pl.ds(c0, 16)] = acc

    for b in range(2):
        start_in(b, in_slots[b])

    def quad_body(q, carry):
        for pb in range(2):
            p = q * 2 + pb
            ob = out_bufs[pb]

            @pl.when(p >= 2)
            def _():
                wait_out(ob)

            for b in range(2):
                g = p * 2 + b
                sl = in_slots[b]
                wait_in(sl)
                compute(sl, ob[0], b)

                @pl.when(g + 2 < 2 * NUM_PAIRS)
                def _():
                    start_in(g + 2, sl)

            start_out(p, ob)
        return carry

    lax.fori_loop(0, NUM_PAIRS // 2, quad_body, 0)

    for pb in range(2):
        wait_out(out_bufs[pb])


_TC_BLOCK_WROWS = 512


def _unpack_body(w_ref, o_ref):
    o_ref[...] = pltpu.bitcast(w_ref[...], jnp.int8)


_unpack_tc = pl.pallas_call(
    _unpack_body,
    grid=(WROWS // _TC_BLOCK_WROWS,),
    in_specs=[pl.BlockSpec((_TC_BLOCK_WROWS, COLS), lambda i: (i, 0))],
    out_specs=pl.BlockSpec((4 * _TC_BLOCK_WROWS, COLS), lambda i: (i, 0)),
    out_shape=jax.ShapeDtypeStruct(SRC_SHAPE, jnp.int8),
)


def kernel(source, rng_seq, rng_idx):
    idx = rng_idx.astype(jnp.int32)
    out_w = _bsgen_sc(source, rng_seq, idx)
    return _unpack_tc(out_w)
